# baseline (speedup 1.0000x reference)
import jax
import jax.numpy as jnp
from jax import lax
from jax.experimental import pallas as pl
from jax.experimental.pallas import tpu as pltpu

B = 16
L = 4096
N = 32768
D_IN = 4
D_MID = 64
D_LOC = 16
D_OUT = 256

NPAD = N + L + 128
TA = 8192
NTA = N // TA
GRID = NTA + B


def _body(feats_ref, ids_ref, W1_ref, b1c_ref, W2_ref, b2c_ref, Wfc_ref,
          bfc_ref, out_ref, mfeat_ref, hbuf, pooled, cnt, offs):
    i = pl.program_id(0)

    @pl.when(i == 0)
    def _():
        ids = ids_ref[...]
        for b in range(B):
            cnt[b] = jnp.sum((ids == b).astype(jnp.int32))
            offs[b] = jnp.sum((ids < b).astype(jnp.int32))

    @pl.when(i < NTA)
    def _():
        f = feats_ref[...]
        hT = jnp.maximum(
            lax.dot_general(W1_ref[...], f, (((0,), (0,)), ((), ())),
                            preferred_element_type=jnp.float32)
            + b1c_ref[...], 0.0)
        hbuf[:, pl.ds(i * TA, TA)] = hT

    @pl.when(i >= NTA)
    def _():
        b = i - NTA
        cb = jnp.minimum(cnt[b], L)
        ob = offs[b]
        obal = (ob // 128) * 128
        d = ob - obal
        win = hbuf[:, pl.ds(obal, L + 128)]
        hchT = pltpu.roll(win, -d, axis=1)[:, 0:L]
        lane_iota = lax.broadcasted_iota(jnp.int32, (1, L), 1)
        mfeat_ref[0] = jnp.where(lane_iota < cb, hchT, 0.0)

        xT = lax.dot_general(W2_ref[...], hchT, (((0,), (0,)), ((), ())),
                             preferred_element_type=jnp.float32) + b2c_ref[...]
        xb = xT.astype(jnp.bfloat16)
        xbm = jnp.where(lane_iota < cb, xb, xb[:, 0:1])
        cols = []
        for jj in range(D_LOC):
            prod = xbm * xbm[jj:jj + 1, :]
            cols.append(jnp.max(prod, axis=1, keepdims=True)
                        .astype(jnp.float32))
        tile = jnp.concatenate(cols, axis=1)
        flat = jnp.concatenate(
            [tile[ii:ii + 1, :] for ii in range(D_LOC)], axis=1)
        flat = jnp.where(cb > 0, flat, jnp.full_like(flat, -1e30))
        pooled[pl.ds(b, 1), :] = flat

    @pl.when(i == GRID - 1)
    def _():
        P = pooled[...]
        pe = jnp.sign(P) * jnp.sqrt(jnp.abs(P) + 1e-8)
        nrm = jnp.sqrt(jnp.sum(pe * pe, axis=1, keepdims=True))
        flatn = pe / (nrm + 1e-12)
        out_ref[...] = jnp.dot(flatn, Wfc_ref[...],
                               preferred_element_type=jnp.float32) + bfc_ref[...]


def kernel(feats, W1, b1, W2, b2, W_fc, b_fc, batch_ids):
    fT = feats.T
    ids2d = batch_ids.astype(jnp.int32).reshape(B, N // B)
    b1c = b1.reshape(D_MID, 1)
    b2c = b2.reshape(D_LOC, 1)
    bfcr = b_fc.reshape(1, D_OUT)

    out, mfeat = pl.pallas_call(
        _body,
        grid=(GRID,),
        in_specs=[
            pl.BlockSpec((D_IN, TA), lambda i: (0, jnp.minimum(i, NTA - 1))),
            pl.BlockSpec((B, N // B), lambda i: (0, 0)),
            pl.BlockSpec((D_IN, D_MID), lambda i: (0, 0)),
            pl.BlockSpec((D_MID, 1), lambda i: (0, 0)),
            pl.BlockSpec((D_MID, D_LOC), lambda i: (0, 0)),
            pl.BlockSpec((D_LOC, 1), lambda i: (0, 0)),
            pl.BlockSpec((D_LOC * D_LOC, D_OUT), lambda i: (0, 0)),
            pl.BlockSpec((1, D_OUT), lambda i: (0, 0)),
        ],
        out_specs=[
            pl.BlockSpec((B, D_OUT), lambda i: (0, 0)),
            pl.BlockSpec(
                (1, D_MID, L),
                lambda i: (jnp.maximum(i - NTA, 0), 0, 0)),
        ],
        out_shape=[
            jax.ShapeDtypeStruct((B, D_OUT), jnp.float32),
            jax.ShapeDtypeStruct((B, D_MID, L), jnp.float32),
        ],
        scratch_shapes=[
            pltpu.VMEM((D_MID, NPAD), jnp.float32),
            pltpu.VMEM((B, D_LOC * D_LOC), jnp.float32),
            pltpu.SMEM((B,), jnp.int32),
            pltpu.SMEM((B,), jnp.int32),
        ],
        compiler_params=pltpu.CompilerParams(
            vmem_limit_bytes=100 * 1024 * 1024),
    )(fT, ids2d, W1, b1c, W2, b2c, W_fc, bfcr)
    return out, jnp.swapaxes(mfeat, 1, 2)


# speedup vs baseline: 1.0239x; 1.0239x over previous
import jax
import jax.numpy as jnp
from jax import lax
from jax.experimental import pallas as pl
from jax.experimental.pallas import tpu as pltpu

B = 16
L = 4096
N = 32768
D_IN = 4
D_MID = 64
D_LOC = 16
D_OUT = 256

NPAD = N + L
TA = 8192
NTA = N // TA
GRID = NTA + B


def _body(feats_ref, ids_ref, W1_ref, b1_ref, W2_ref, b2c_ref, Wfc_ref,
          bfc_ref, out_ref, mfeat_ref, hbuf, pooled, cnt, offs):
    i = pl.program_id(0)

    @pl.when(i == 0)
    def _():
        ids = ids_ref[...]
        for b in range(B):
            cnt[b] = jnp.sum((ids == b).astype(jnp.int32))
            offs[b] = jnp.sum((ids < b).astype(jnp.int32))

    @pl.when(i < NTA)
    def _():
        f = feats_ref[...]
        h = jnp.maximum(
            lax.dot_general(f, W1_ref[...], (((0,), (0,)), ((), ())),
                            preferred_element_type=jnp.float32)
            + b1_ref[...], 0.0)
        hbuf[pl.ds(i * TA, TA), :] = h

    @pl.when(i >= NTA)
    def _():
        b = i - NTA
        cb = jnp.minimum(cnt[b], L)
        ob = offs[b]
        hch = hbuf[pl.ds(ob, L), :]
        hchT = jnp.transpose(hch, (1, 0))
        lane_iota = lax.broadcasted_iota(jnp.int32, (1, L), 1)
        mfeat_ref[0] = jnp.where(lane_iota < cb, hchT, 0.0)

        xT = lax.dot_general(W2_ref[...], hchT, (((0,), (0,)), ((), ())),
                             preferred_element_type=jnp.float32) + b2c_ref[...]
        xb = xT.astype(jnp.bfloat16)
        xbm = jnp.where(lane_iota < cb, xb, xb[:, 0:1])
        cols = []
        for jj in range(D_LOC):
            prod = xbm * xbm[jj:jj + 1, :]
            cols.append(jnp.max(prod, axis=1, keepdims=True)
                        .astype(jnp.float32))
        tile = jnp.concatenate(cols, axis=1)
        flat = jnp.concatenate(
            [tile[ii:ii + 1, :] for ii in range(D_LOC)], axis=1)
        flat = jnp.where(cb > 0, flat, jnp.full_like(flat, -1e30))
        pooled[pl.ds(b, 1), :] = flat

    @pl.when(i == GRID - 1)
    def _():
        P = pooled[...]
        pe = jnp.sign(P) * jnp.sqrt(jnp.abs(P) + 1e-8)
        nrm = jnp.sqrt(jnp.sum(pe * pe, axis=1, keepdims=True))
        flatn = pe / (nrm + 1e-12)
        out_ref[...] = jnp.dot(flatn, Wfc_ref[...],
                               preferred_element_type=jnp.float32) + bfc_ref[...]


def kernel(feats, W1, b1, W2, b2, W_fc, b_fc, batch_ids):
    fT = feats.T
    ids2d = batch_ids.astype(jnp.int32).reshape(B, N // B)
    b1r = b1.reshape(1, D_MID)
    b2c = b2.reshape(D_LOC, 1)
    bfcr = b_fc.reshape(1, D_OUT)

    out, mfeat = pl.pallas_call(
        _body,
        grid=(GRID,),
        in_specs=[
            pl.BlockSpec((D_IN, TA), lambda i: (0, jnp.minimum(i, NTA - 1))),
            pl.BlockSpec((B, N // B), lambda i: (0, 0)),
            pl.BlockSpec((D_IN, D_MID), lambda i: (0, 0)),
            pl.BlockSpec((1, D_MID), lambda i: (0, 0)),
            pl.BlockSpec((D_MID, D_LOC), lambda i: (0, 0)),
            pl.BlockSpec((D_LOC, 1), lambda i: (0, 0)),
            pl.BlockSpec((D_LOC * D_LOC, D_OUT), lambda i: (0, 0)),
            pl.BlockSpec((1, D_OUT), lambda i: (0, 0)),
        ],
        out_specs=[
            pl.BlockSpec((B, D_OUT), lambda i: (0, 0)),
            pl.BlockSpec(
                (1, D_MID, L),
                lambda i: (jnp.maximum(i - NTA, 0), 0, 0)),
        ],
        out_shape=[
            jax.ShapeDtypeStruct((B, D_OUT), jnp.float32),
            jax.ShapeDtypeStruct((B, D_MID, L), jnp.float32),
        ],
        scratch_shapes=[
            pltpu.VMEM((NPAD, D_MID), jnp.float32),
            pltpu.VMEM((B, D_LOC * D_LOC), jnp.float32),
            pltpu.SMEM((B,), jnp.int32),
            pltpu.SMEM((B,), jnp.int32),
        ],
        compiler_params=pltpu.CompilerParams(
            vmem_limit_bytes=100 * 1024 * 1024),
    )(fT, ids2d, W1, b1r, W2, b2c, W_fc, bfcr)
    return out, jnp.swapaxes(mfeat, 1, 2)
